# baseline (speedup 1.0000x reference)
import jax
import jax.numpy as jnp
from jax.experimental import pallas as pl

B, F, D, N, C, NEIGH = 8, 256, 128, 512, 4, 32
FB = 16


def _bmm_bf16(a, b):
    return jax.lax.dot_general(
        a.astype(jnp.bfloat16), b.astype(jnp.bfloat16),
        (((2,), (1,)), ((0,), (0,))), preferred_element_type=jnp.float32)


def _mlp_body(x_ref, w1_ref, b1_ref, w2_ref, b2_ref, out_ref):
    w1n = w1_ref[...]
    w2n = w2_ref[...]
    x = x_ref[...]
    h = _bmm_bf16(x, w1n) + b1_ref[...][:, None, :]
    h = 0.5 * h * (1.0 + jax.lax.erf(h / (2.0 ** 0.5)))
    out_ref[...] = _bmm_bf16(h, w2n) + b2_ref[...][:, None, :]


def _edge_body(nf_ref, qkw_ref, edge_ref):
    nf = nf_ref[0]
    qkw = qkw_ref[...]
    qk = jnp.dot(nf, qkw, preferred_element_type=jnp.float32)
    scale = F ** -0.5
    attns = []
    se = jnp.zeros((N, N), jnp.float32)
    for c in range(C):
        q = qk[:, c * F:(c + 1) * F]
        k = qk[:, (C + c) * F:(C + c + 1) * F]
        lg = jax.lax.dot_general(q, k, (((1,), (1,)), ((), ())),
                                 preferred_element_type=jnp.float32) * scale
        m = jnp.max(lg, axis=1, keepdims=True)
        ex = jnp.exp(lg - m)
        at = ex / jnp.sum(ex, axis=1, keepdims=True)
        attns.append(at)
        se = se + at

    bits = jax.lax.bitcast_convert_type(se, jnp.int32)

    def bis(_, carry):
        lo, hi = carry
        mid = lo + ((hi - lo) >> 1)
        cnt = jnp.sum((bits >= mid).astype(jnp.int32), axis=1, keepdims=True)
        ge = cnt >= NEIGH
        return jnp.where(ge, mid, lo), jnp.where(ge, hi, mid)

    lo = jnp.zeros((N, 1), jnp.int32)
    hi = jnp.full((N, 1), 0x7f800000, jnp.int32)
    lo, hi = jax.lax.fori_loop(0, 32, bis, (lo, hi))

    row = jax.lax.broadcasted_iota(jnp.int32, (N, N), 0)
    col = jax.lax.broadcasted_iota(jnp.int32, (N, N), 1)
    gt = bits > lo
    eq = bits == lo
    eqf = eq.astype(jnp.float32)
    need = (NEIGH - jnp.sum(gt.astype(jnp.int32), axis=1, keepdims=True)
            ).astype(jnp.float32)
    slt = (row < col).astype(jnp.float32)
    cumex = jax.lax.dot_general(eqf, slt, (((1,), (0,)), ((), ())),
                                preferred_element_type=jnp.float32)
    mask = jnp.where(gt | (eq & (cumex < need)) | (row == col), 1.0, 0.0)

    for c in range(C):
        e = mask * attns[c]
        nr = e / (jnp.sum(e, axis=1, keepdims=True) + 1e-6)
        nc = nr / (jnp.sum(nr, axis=0, keepdims=True) + 1e-6)
        edge_ref[0, c] = jax.lax.dot_general(
            nr, nc, (((1,), (1,)), ((), ())), preferred_element_type=jnp.float32)


def _spectral_normalize_xla(w):
    s = jnp.linalg.svd(w, compute_uv=False)[..., 0]
    s = jnp.maximum(s, 1e-6)
    return w / jax.lax.stop_gradient(s)[:, None, None]


def kernel(inputs, weight1, bias1, weight2, bias2, qk_weight):
    w1n = _spectral_normalize_xla(weight1)
    w2n = _spectral_normalize_xla(weight2)
    x_t = jnp.swapaxes(inputs, 0, 1)
    b1 = bias1[:, 0, :]
    b2 = bias2[:, 0, :]

    conv = pl.pallas_call(
        _mlp_body,
        grid=(F // FB,),
        in_specs=[
            pl.BlockSpec((FB, B, D), lambda i: (i, 0, 0)),
            pl.BlockSpec((FB, D, D), lambda i: (i, 0, 0)),
            pl.BlockSpec((FB, D), lambda i: (i, 0)),
            pl.BlockSpec((FB, D, N), lambda i: (i, 0, 0)),
            pl.BlockSpec((FB, N), lambda i: (i, 0)),
        ],
        out_specs=pl.BlockSpec((FB, B, N), lambda i: (i, 0, 0)),
        out_shape=jax.ShapeDtypeStruct((F, B, N), jnp.float32),
    )(x_t, w1n, b1, w2n, b2)

    node_features = jnp.transpose(conv, (1, 2, 0))
    qkw_t = qk_weight.T

    edge = pl.pallas_call(
        _edge_body,
        grid=(B,),
        in_specs=[
            pl.BlockSpec((1, N, F), lambda b: (b, 0, 0)),
            pl.BlockSpec((F, 2 * C * F), lambda b: (0, 0)),
        ],
        out_specs=pl.BlockSpec((1, C, N, N), lambda b: (b, 0, 0, 0)),
        out_shape=jax.ShapeDtypeStruct((B, C, N, N), jnp.float32),
    )(node_features, qkw_t)

    return node_features, edge
